# baseline (speedup 1.0000x reference)
import jax
import jax.numpy as jnp
from jax.experimental import pallas as pl
from jax.experimental.pallas import tpu as pltpu

_N = 5000
_C = 256
_HW = 49
_HIDDEN = 1024
_NCLS = 81
_NBOX = 324
_EPS = 1e-5

_TN = 40
_NB = _N // _TN
_CH = 1000
_NC = _N // _CH


def _head_kernel(x_ref, w1_ref, b1_ref, g_ref, be_ref,
                 wc_ref, bc_ref, wb_ref, bb_ref,
                 oc_ref, ob_ref, h_s, s_s):
    i = pl.program_id(0)

    @pl.when(i < _NB)
    def _phase0():
        xs = jnp.sum(x_ref[...], axis=2)
        hb = (jnp.dot(xs, w1_ref[...], preferred_element_type=jnp.float32)
              + b1_ref[...])
        h_s[pl.ds(i * _TN, _TN), :] = hb
        p1 = jnp.sum(hb, axis=0, keepdims=True)
        p2 = jnp.sum(hb * hb, axis=0, keepdims=True)

        @pl.when(i == 0)
        def _():
            s_s[0:1, :] = p1
            s_s[1:2, :] = p2

        @pl.when(i > 0)
        def _():
            s_s[0:1, :] = s_s[0:1, :] + p1
            s_s[1:2, :] = s_s[1:2, :] + p2

    @pl.when(i >= _NB)
    def _phase1():
        c = i - _NB
        mean = s_s[0:1, :] * (1.0 / _N)
        var = s_s[1:2, :] * (1.0 / _N) - mean * mean
        inv = jax.lax.rsqrt(var + _EPS)
        scale = g_ref[...] * inv
        shift = be_ref[...] - mean * scale
        hb = h_s[pl.ds(c * _CH, _CH), :]
        y = jnp.maximum(hb * scale + shift, 0.0)
        oc_ref[...] = (jnp.dot(y, wc_ref[...], preferred_element_type=jnp.float32)
                       + bc_ref[...])
        ob_ref[...] = (jnp.dot(y, wb_ref[...], preferred_element_type=jnp.float32)
                       + bb_ref[...])


def kernel(x, fc1_w, fc1_b, bn_gamma, bn_beta, cls_w, cls_b, box_w, box_b):
    x_r = x.reshape(_N, _C, _HW)
    w1 = fc1_w.T * (1.0 / _HW)
    wc = cls_w.T
    wb = box_w.T
    b1 = fc1_b.reshape(1, _HIDDEN)
    g = bn_gamma.reshape(1, _HIDDEN)
    be = bn_beta.reshape(1, _HIDDEN)
    bc = cls_b.reshape(1, _NCLS)
    bb = box_b.reshape(1, _NBOX)

    last0 = _NB - 1
    grid = (_NB + _NC,)

    out_cls, out_box = pl.pallas_call(
        _head_kernel,
        grid=grid,
        in_specs=[
            pl.BlockSpec((_TN, _C, _HW),
                         lambda i: (jnp.minimum(i, last0), 0, 0)),
            pl.BlockSpec((_C, _HIDDEN), lambda i: (0, 0)),
            pl.BlockSpec((1, _HIDDEN), lambda i: (0, 0)),
            pl.BlockSpec((1, _HIDDEN), lambda i: (0, 0)),
            pl.BlockSpec((1, _HIDDEN), lambda i: (0, 0)),
            pl.BlockSpec((_HIDDEN, _NCLS), lambda i: (0, 0)),
            pl.BlockSpec((1, _NCLS), lambda i: (0, 0)),
            pl.BlockSpec((_HIDDEN, _NBOX), lambda i: (0, 0)),
            pl.BlockSpec((1, _NBOX), lambda i: (0, 0)),
        ],
        out_specs=[
            pl.BlockSpec((_CH, _NCLS), lambda i: (jnp.maximum(i - _NB, 0), 0)),
            pl.BlockSpec((_CH, _NBOX), lambda i: (jnp.maximum(i - _NB, 0), 0)),
        ],
        out_shape=[
            jax.ShapeDtypeStruct((_N, _NCLS), jnp.float32),
            jax.ShapeDtypeStruct((_N, _NBOX), jnp.float32),
        ],
        scratch_shapes=[
            pltpu.VMEM((_N, _HIDDEN), jnp.float32),
            pltpu.VMEM((2, _HIDDEN), jnp.float32),
        ],
        compiler_params=pltpu.CompilerParams(
            dimension_semantics=("arbitrary",),
        ),
    )(x_r, w1, b1, g, be, wc, bc, wb, bb)

    return (out_cls, out_box)
